# baseline (speedup 1.0000x reference)
import jax
import jax.numpy as jnp
from jax import lax
from jax.experimental import pallas as pl
from jax.experimental.pallas import tpu as pltpu
from jax.experimental.pallas import tpu_sc as plsc

_B, _NH, _T = 4, 16, 8192
_S = 256
_K = 5
_NSUB = 16
_NCORE = 2
_CPT = _T // _NSUB
_BPC = _B // _NCORE
_ROWS = _CPT // 128

def _vsum(v):
    return plsc.cumsum(v)[15]


def _vmax(v):
    return plsc.cummax(v)[15]


def _vmin_i32(v):
    return -plsc.cummax(-v)[15]


_mesh = plsc.VectorSubcoreMesh(
    core_axis_name="c", subcore_axis_name="s",
    num_cores=_NCORE, num_subcores=_NSUB,
)


def _sc_body(attn, map2d, zeros, vals_out, idx_out,
             att_l, map_l, tv0, tv1, fa0, fa1, vout, iout, acc0, acc1):
    cid = lax.axis_index("c")
    sid = lax.axis_index("s")
    base = sid * _CPT
    lanes = jnp.arange(16, dtype=jnp.int32)

    @pl.when(sid == 0)
    def _init():
        pltpu.sync_copy(zeros, acc0)
        pltpu.sync_copy(zeros, acc1)

    pltpu.sync_copy(
        attn.at[pl.ds(cid * (_BPC * _NH), _BPC * _NH), pl.ds(base, _CPT)],
        att_l)
    pltpu.sync_copy(map2d.at[pl.ds(sid * _ROWS, _ROWS)], map_l)

    def g_body(g, carry):
        t0 = g * 16
        for b, tv in ((0, tv0), (1, tv1)):
            s = att_l[b * _NH, pl.ds(t0, 16)]
            for h in range(1, _NH):
                s = s + att_l[b * _NH + h, pl.ds(t0, 16)]
            tv[pl.ds(t0, 16)] = s * jnp.float32(1.0 / _NH)
        return carry

    lax.fori_loop(0, _CPT // 16, g_body, 0)

    plsc.subcore_barrier()
    for tv, acc in ((tv0, acc0), (tv1, acc1)):
        for j in range(_ROWS):
            pltpu.sync_copy(tv.at[pl.ds(j * 128, 128)],
                            acc.at[map_l.at[j]], add=True)
    plsc.subcore_barrier()

    @pl.when(sid == 0)
    def _final():
        pltpu.sync_copy(acc0, fa0)
        pltpu.sync_copy(acc1, fa1)
        for b, fa in ((0, fa0), (1, fa1)):
            vecs = [fa[pl.ds(i * 16, 16)] for i in range(_S // 16)]
            tot_v = vecs[0]
            for v in vecs[1:]:
                tot_v = tot_v + v
            tot = _vsum(tot_v)
            vecs = [v / tot for v in vecs]
            vvec = jnp.zeros((16,), jnp.float32)
            ivec = jnp.zeros((16,), jnp.int32)
            for i in range(_K):
                m = vecs[0]
                for v in vecs[1:]:
                    m = jnp.maximum(m, v)
                mx = _vmax(m)
                cm = jnp.where(vecs[0] == mx, lanes, jnp.int32(1 << 30))
                for j in range(1, _S // 16):
                    cj = jnp.where(vecs[j] == mx, j * 16 + lanes,
                                   jnp.int32(1 << 30))
                    cm = jnp.minimum(cm, cj)
                ind = _vmin_i32(cm)
                vvec = jnp.where(lanes == i, mx, vvec)
                ivec = jnp.where(lanes == i, ind, ivec)
                vecs = [jnp.where(j * 16 + lanes == ind, jnp.float32(-1.0),
                                  vecs[j]) for j in range(_S // 16)]
            vout[pl.ds(b * 16, 16)] = vvec
            iout[pl.ds(b * 16, 16)] = ivec
        pltpu.sync_copy(vout, vals_out.at[pl.ds(cid * (_BPC * 16),
                                                _BPC * 16)])
        pltpu.sync_copy(iout, idx_out.at[pl.ds(cid * (_BPC * 16),
                                               _BPC * 16)])


def kernel(attention_weights, token_to_sentence_map):
    attn2d = attention_weights.reshape(_B * _NH, _T)
    map2d = token_to_sentence_map.astype(jnp.int32).reshape(_T // 128, 128)
    zeros = jnp.zeros((_S,), jnp.float32)
    run = pl.kernel(
        _sc_body,
        out_type=[
            jax.ShapeDtypeStruct((_B * 16,), jnp.float32),
            jax.ShapeDtypeStruct((_B * 16,), jnp.int32),
        ],
        mesh=_mesh,
        compiler_params=pltpu.CompilerParams(needs_layout_passes=False),
        scratch_types=[
            pltpu.VMEM((_BPC * _NH, _CPT), jnp.float32),
            pltpu.VMEM((_ROWS, 128), jnp.int32),
            pltpu.VMEM((_CPT,), jnp.float32),
            pltpu.VMEM((_CPT,), jnp.float32),
            pltpu.VMEM((_S,), jnp.float32),
            pltpu.VMEM((_S,), jnp.float32),
            pltpu.VMEM((_BPC * 16,), jnp.float32),
            pltpu.VMEM((_BPC * 16,), jnp.int32),
            pltpu.VMEM_SHARED((_S,), jnp.float32),
            pltpu.VMEM_SHARED((_S,), jnp.float32),
        ],
    )
    vals, idx = run(attn2d, map2d, zeros)
    return vals.reshape(_B, 16)[:, :_K], idx.reshape(_B, 16)[:, :_K]


# speedup vs baseline: 1.0891x; 1.0891x over previous
import jax
import jax.numpy as jnp
from jax import lax
from jax.experimental import pallas as pl
from jax.experimental.pallas import tpu as pltpu
from jax.experimental.pallas import tpu_sc as plsc

_B, _NH, _T = 4, 16, 8192
_S = 256
_K = 5
_NSUB = 16
_NCORE = 2
_CPT = _T // _NSUB
_BPC = _B // _NCORE
_ROWS = _CPT // 128

_mesh = plsc.VectorSubcoreMesh(
    core_axis_name="c", subcore_axis_name="s",
    num_cores=_NCORE, num_subcores=_NSUB,
)


def _sc_body(attn, map2d, part, att_l, map_l, tv0, tv1, zb, acc0, acc1):
    cid = lax.axis_index("c")
    sid = lax.axis_index("s")
    base = sid * _CPT

    @pl.when(sid == 0)
    def _init():
        for i in range(_S // 16):
            zb[pl.ds(i * 16, 16)] = jnp.zeros((16,), jnp.float32)
        pltpu.sync_copy(zb, acc0)
        pltpu.sync_copy(zb, acc1)

    pltpu.sync_copy(
        attn.at[pl.ds(cid * (_BPC * _NH), _BPC * _NH), pl.ds(base, _CPT)],
        att_l)
    pltpu.sync_copy(map2d.at[pl.ds(sid * _ROWS, _ROWS)], map_l)

    def g_body(g, carry):
        t0 = g * 16
        for b, tv in ((0, tv0), (1, tv1)):
            s = att_l[b * _NH, pl.ds(t0, 16)]
            for h in range(1, _NH):
                s = s + att_l[b * _NH + h, pl.ds(t0, 16)]
            tv[pl.ds(t0, 16)] = s * jnp.float32(1.0 / _NH)
        return carry

    lax.fori_loop(0, _CPT // 16, g_body, 0)

    plsc.subcore_barrier()
    for tv, acc in ((tv0, acc0), (tv1, acc1)):
        for j in range(_ROWS):
            pltpu.sync_copy(tv.at[pl.ds(j * 128, 128)],
                            acc.at[map_l.at[j]], add=True)
    plsc.subcore_barrier()

    @pl.when(sid == 0)
    def _out():
        pltpu.sync_copy(acc0, part.at[cid * _BPC])
        pltpu.sync_copy(acc1, part.at[cid * _BPC + 1])


def _tc_body(part_ref, vals_ref, idx_ref):
    scores = part_ref[...]
    total = jnp.sum(scores, axis=-1, keepdims=True)
    work = scores / total
    col = lax.broadcasted_iota(jnp.int32, (_B, 8), 1)
    sent = lax.broadcasted_iota(jnp.int32, (_B, _S), 1)
    vals_acc = jnp.zeros((_B, 8), jnp.float32)
    idx_acc = jnp.zeros((_B, 8), jnp.int32)
    for i in range(_K):
        mx = jnp.max(work, axis=-1, keepdims=True)
        cand = jnp.where(work == mx, sent, jnp.int32(1 << 30))
        ind = jnp.min(cand, axis=-1, keepdims=True)
        vals_acc = jnp.where(col == i, mx, vals_acc)
        idx_acc = jnp.where(col == i, ind, idx_acc)
        work = jnp.where(sent == ind, jnp.float32(-1.0), work)
    vals_ref[...] = vals_acc[:, :_K]
    idx_ref[...] = idx_acc[:, :_K]


def kernel(attention_weights, token_to_sentence_map):
    attn2d = attention_weights.reshape(_B * _NH, _T)
    map2d = token_to_sentence_map.astype(jnp.int32).reshape(_T // 128, 128)
    sc_run = pl.kernel(
        _sc_body,
        out_type=jax.ShapeDtypeStruct((_B, _S), jnp.float32),
        mesh=_mesh,
        compiler_params=pltpu.CompilerParams(needs_layout_passes=False),
        scratch_types=[
            pltpu.VMEM((_BPC * _NH, _CPT), jnp.float32),
            pltpu.VMEM((_ROWS, 128), jnp.int32),
            pltpu.VMEM((_CPT,), jnp.float32),
            pltpu.VMEM((_CPT,), jnp.float32),
            pltpu.VMEM((_S,), jnp.float32),
            pltpu.VMEM_SHARED((_S,), jnp.float32),
            pltpu.VMEM_SHARED((_S,), jnp.float32),
        ],
    )
    part = sc_run(attn2d, map2d)
    vals, idx = pl.pallas_call(
        _tc_body,
        out_shape=[
            jax.ShapeDtypeStruct((_B, _K), jnp.float32),
            jax.ShapeDtypeStruct((_B, _K), jnp.int32),
        ],
    )(part)
    return vals, idx
